# baseline (device time: 36024 ns/iter reference)
import jax
import jax.numpy as jnp
from jax import lax
from jax.experimental import pallas as pl
from jax.experimental.pallas import tpu as pltpu

N_DEV = 8


def kernel(x, Win0, Wout0, Win1, Wout1, Win2, Wout2):
    M, D = x.shape
    F = Win0.shape[1]
    CH = M // N_DEV
    bf16 = jnp.bfloat16

    def body(x_ref, win0, wout0, win1, wout1, win2, wout2, out_ref,
             xbuf, partial, rsbuf, win_bf, wout_bf,
             rs_send, rs_recv, ag_send, ag_recv):
        me = lax.axis_index("i")

        barrier = pltpu.get_barrier_semaphore()
        for p in range(N_DEV):
            @pl.when(me != p)
            def _(p=p):
                pl.semaphore_signal(
                    barrier, inc=1,
                    device_id=(p,), device_id_type=pl.DeviceIdType.MESH,
                )
        pl.semaphore_wait(barrier, N_DEV - 1)

        def rs_copy(p):
            return pltpu.make_async_remote_copy(
                src_ref=partial.at[pl.ds(p * CH, CH), :],
                dst_ref=rsbuf.at[pl.ds(me * CH, CH), :],
                send_sem=rs_send.at[p],
                recv_sem=rs_recv.at[me],
                device_id=(p,),
                device_id_type=pl.DeviceIdType.MESH,
            )

        def rs_wait_from(s):
            return pltpu.make_async_remote_copy(
                src_ref=partial.at[pl.ds(0, CH), :],
                dst_ref=rsbuf.at[pl.ds(s * CH, CH), :],
                send_sem=rs_send.at[s],
                recv_sem=rs_recv.at[s],
                device_id=(s,),
                device_id_type=pl.DeviceIdType.MESH,
            )

        def ag_copy(p):
            return pltpu.make_async_remote_copy(
                src_ref=xbuf.at[pl.ds(me * CH, CH), :],
                dst_ref=xbuf.at[pl.ds(me * CH, CH), :],
                send_sem=ag_send.at[p],
                recv_sem=ag_recv.at[me],
                device_id=(p,),
                device_id_type=pl.DeviceIdType.MESH,
            )

        def ag_wait_from(s):
            return pltpu.make_async_remote_copy(
                src_ref=xbuf.at[pl.ds(0, CH), :],
                dst_ref=xbuf.at[pl.ds(s * CH, CH), :],
                send_sem=ag_send.at[s],
                recv_sem=ag_recv.at[s],
                device_id=(s,),
                device_id_type=pl.DeviceIdType.MESH,
            )

        wins = [win0, win1, win2]
        wouts = [wout0, wout1, wout2]

        def load_weights(l):
            win_bf[...] = wins[l][...].astype(bf16)
            wout_bf[...] = wouts[l][...].astype(bf16)

        def compute_rows(xv):
            h = jnp.dot(xv, win_bf[...], preferred_element_type=jnp.float32)
            h = jnp.maximum(h, 0.0).astype(bf16)
            pv = jnp.dot(h, wout_bf[...], preferred_element_type=jnp.float32)
            return pv.astype(bf16)

        def reduce_own_chunk():
            acc = partial[pl.ds(me * CH, CH), :].astype(jnp.float32)
            for s in range(N_DEV):
                contrib = rsbuf[pl.ds(s * CH, CH), :].astype(jnp.float32)
                acc = acc + jnp.where(s == me, 0.0, contrib)
            return acc.astype(bf16)

        with jax.named_scope("layer0_compute"):
            load_weights(0)
            partial[...] = compute_rows(x_ref[...].astype(bf16))
        with jax.named_scope("layer0_rs"):
            for p in range(N_DEV):
                @pl.when(me != p)
                def _(p=p):
                    rs_copy(p).start()
            for s in range(N_DEV):
                @pl.when(me != s)
                def _(s=s):
                    rs_wait_from(s).wait_recv()
        with jax.named_scope("layer0_reduce_ag"):
            xbuf[pl.ds(me * CH, CH), :] = reduce_own_chunk()
            for p in range(N_DEV):
                @pl.when(me != p)
                def _(p=p):
                    ag_copy(p).start()

        for l in (1, 2):
            with jax.named_scope(f"layer{l}_fused"):
                load_weights(l)
                for p in range(N_DEV):
                    @pl.when(me != p)
                    def _(p=p):
                        rs_copy(p).wait_send()
                partial[pl.ds(me * CH, CH), :] = compute_rows(
                    xbuf[pl.ds(me * CH, CH), :])
                for s in range(N_DEV):
                    @pl.when(me != s)
                    def _(s=s):
                        ag_wait_from(s).wait_recv()
                        partial[s * CH:(s + 1) * CH, :] = compute_rows(
                            xbuf[s * CH:(s + 1) * CH, :])
                        rs_copy(s).start()
            with jax.named_scope(f"layer{l}_rs_wait"):
                for s in range(N_DEV):
                    @pl.when(me != s)
                    def _(s=s):
                        rs_wait_from(s).wait_recv()
            with jax.named_scope(f"layer{l}_reduce_ag"):
                red = reduce_own_chunk()
                for p in range(N_DEV):
                    @pl.when(me != p)
                    def _(p=p):
                        ag_copy(p).wait_send()
                xbuf[pl.ds(me * CH, CH), :] = red
                for p in range(N_DEV):
                    @pl.when(me != p)
                    def _(p=p):
                        ag_copy(p).start()

        with jax.named_scope("final_gather"):
            for s in range(N_DEV):
                @pl.when(me != s)
                def _(s=s):
                    ag_wait_from(s).wait_recv()
            out_ref[...] = xbuf[...].astype(jnp.float32)
            for p in range(N_DEV):
                @pl.when(me != p)
                def _(p=p):
                    rs_copy(p).wait_send()
                    ag_copy(p).wait_send()

    return pl.pallas_call(
        body,
        out_shape=jax.ShapeDtypeStruct((M, D), jnp.float32),
        in_specs=[pl.BlockSpec(memory_space=pltpu.VMEM)] * 7,
        out_specs=pl.BlockSpec(memory_space=pltpu.VMEM),
        scratch_shapes=[
            pltpu.VMEM((M, D), bf16),
            pltpu.VMEM((M, D), bf16),
            pltpu.VMEM((M, D), bf16),
            pltpu.VMEM((D, F), bf16),
            pltpu.VMEM((F, D), bf16),
            pltpu.SemaphoreType.DMA((N_DEV,)),
            pltpu.SemaphoreType.DMA((N_DEV,)),
            pltpu.SemaphoreType.DMA((N_DEV,)),
            pltpu.SemaphoreType.DMA((N_DEV,)),
        ],
        compiler_params=pltpu.CompilerParams(collective_id=0),
    )(x, Win0, Wout0, Win1, Wout1, Win2, Wout2)


# device time: 35590 ns/iter; 1.0122x vs baseline; 1.0122x over previous
import jax
import jax.numpy as jnp
from jax import lax
from jax.experimental import pallas as pl
from jax.experimental.pallas import tpu as pltpu

N_DEV = 8


def kernel(x, Win0, Wout0, Win1, Wout1, Win2, Wout2):
    M, D = x.shape
    F = Win0.shape[1]
    CH = M // N_DEV
    bf16 = jnp.bfloat16

    def body(x_ref, win0, wout0, win1, wout1, win2, wout2, out_ref,
             xbuf, partial, rsbuf, win_bf, wout_bf,
             rs_send, rs_recv, ag_send, ag_recv):
        me = lax.axis_index("i")

        barrier = pltpu.get_barrier_semaphore()
        for p in range(N_DEV):
            @pl.when(me != p)
            def _(p=p):
                pl.semaphore_signal(
                    barrier, inc=1,
                    device_id=(p,), device_id_type=pl.DeviceIdType.MESH,
                )
        pl.semaphore_wait(barrier, N_DEV - 1)

        def rs_copy(p):
            return pltpu.make_async_remote_copy(
                src_ref=partial.at[pl.ds(p * CH, CH), :],
                dst_ref=rsbuf.at[pl.ds(me * CH, CH), :],
                send_sem=rs_send.at[p],
                recv_sem=rs_recv.at[me],
                device_id=(p,),
                device_id_type=pl.DeviceIdType.MESH,
            )

        def rs_wait_from(s):
            return pltpu.make_async_remote_copy(
                src_ref=partial.at[pl.ds(0, CH), :],
                dst_ref=rsbuf.at[pl.ds(s * CH, CH), :],
                send_sem=rs_send.at[s],
                recv_sem=rs_recv.at[s],
                device_id=(s,),
                device_id_type=pl.DeviceIdType.MESH,
            )

        def ag_copy(p):
            return pltpu.make_async_remote_copy(
                src_ref=xbuf.at[pl.ds(me * CH, CH), :],
                dst_ref=xbuf.at[pl.ds(me * CH, CH), :],
                send_sem=ag_send.at[p],
                recv_sem=ag_recv.at[me],
                device_id=(p,),
                device_id_type=pl.DeviceIdType.MESH,
            )

        def ag_wait_from(s):
            return pltpu.make_async_remote_copy(
                src_ref=xbuf.at[pl.ds(0, CH), :],
                dst_ref=xbuf.at[pl.ds(s * CH, CH), :],
                send_sem=ag_send.at[s],
                recv_sem=ag_recv.at[s],
                device_id=(s,),
                device_id_type=pl.DeviceIdType.MESH,
            )

        wins = [win0, win1, win2]
        wouts = [wout0, wout1, wout2]

        def load_weights(l):
            win_bf[...] = wins[l][...].astype(bf16)
            wout_bf[...] = wouts[l][...].astype(bf16)

        def compute_rows(xv):
            h = jnp.dot(xv, win_bf[...], preferred_element_type=jnp.float32)
            h = jnp.maximum(h, 0.0).astype(bf16)
            pv = jnp.dot(h, wout_bf[...], preferred_element_type=jnp.float32)
            return pv.astype(bf16)

        def reduce_own_chunk():
            acc = partial[pl.ds(me * CH, CH), :].astype(jnp.float32)
            for s in range(N_DEV):
                contrib = rsbuf[pl.ds(s * CH, CH), :].astype(jnp.float32)
                acc = acc + jnp.where(s == me, 0.0, contrib)
            return acc.astype(bf16)

        with jax.named_scope("layer0_compute"):
            load_weights(0)
            partial[...] = compute_rows(x_ref[...].astype(bf16))
        with jax.named_scope("layer0_rs"):
            for p in range(N_DEV):
                @pl.when(me != p)
                def _(p=p):
                    rs_copy(p).start()
            for s in range(N_DEV):
                @pl.when(me != s)
                def _(s=s):
                    rs_wait_from(s).wait_recv()
        with jax.named_scope("layer0_reduce_ag"):
            xbuf[pl.ds(me * CH, CH), :] = reduce_own_chunk()
            for p in range(N_DEV):
                @pl.when(me != p)
                def _(p=p):
                    ag_copy(p).start()

        HALF = N_DEV // 2
        for l in (1, 2):
            with jax.named_scope(f"layer{l}_fused"):
                load_weights(l)
                for p in range(N_DEV):
                    @pl.when(me != p)
                    def _(p=p):
                        rs_copy(p).wait_send()
                for g in range(2):
                    lo = g * HALF
                    for s in range(lo, lo + HALF):
                        @pl.when(me != s)
                        def _(s=s):
                            ag_wait_from(s).wait_recv()
                    rows = slice(lo * CH, (lo + HALF) * CH)
                    partial[rows, :] = compute_rows(xbuf[rows, :])
                    for p in range(lo, lo + HALF):
                        @pl.when(me != p)
                        def _(p=p):
                            rs_copy(p).start()
            with jax.named_scope(f"layer{l}_rs_wait"):
                for s in range(N_DEV):
                    @pl.when(me != s)
                    def _(s=s):
                        rs_wait_from(s).wait_recv()
            with jax.named_scope(f"layer{l}_reduce_ag"):
                red = reduce_own_chunk()
                for p in range(N_DEV):
                    @pl.when(me != p)
                    def _(p=p):
                        ag_copy(p).wait_send()
                xbuf[pl.ds(me * CH, CH), :] = red
                for p in range(N_DEV):
                    @pl.when(me != p)
                    def _(p=p):
                        ag_copy(p).start()

        with jax.named_scope("final_gather"):
            for s in range(N_DEV):
                @pl.when(me != s)
                def _(s=s):
                    ag_wait_from(s).wait_recv()
            out_ref[...] = xbuf[...].astype(jnp.float32)
            for p in range(N_DEV):
                @pl.when(me != p)
                def _(p=p):
                    rs_copy(p).wait_send()
                    ag_copy(p).wait_send()

    return pl.pallas_call(
        body,
        out_shape=jax.ShapeDtypeStruct((M, D), jnp.float32),
        in_specs=[pl.BlockSpec(memory_space=pltpu.VMEM)] * 7,
        out_specs=pl.BlockSpec(memory_space=pltpu.VMEM),
        scratch_shapes=[
            pltpu.VMEM((M, D), bf16),
            pltpu.VMEM((M, D), bf16),
            pltpu.VMEM((M, D), bf16),
            pltpu.VMEM((D, F), bf16),
            pltpu.VMEM((F, D), bf16),
            pltpu.SemaphoreType.DMA((N_DEV,)),
            pltpu.SemaphoreType.DMA((N_DEV,)),
            pltpu.SemaphoreType.DMA((N_DEV,)),
            pltpu.SemaphoreType.DMA((N_DEV,)),
        ],
        compiler_params=pltpu.CompilerParams(collective_id=0),
    )(x, Win0, Wout0, Win1, Wout1, Win2, Wout2)
